# baseline (speedup 1.0000x reference)
import functools
import math

import jax
import jax.numpy as jnp
from jax import lax
from jax.experimental import pallas as pl
from jax.experimental.pallas import tpu as pltpu
from jax.experimental.pallas import tpu_sc as plsc

_VOCAB = 100000
_D = 512
_SEQ = 8192

_NC = 2
_NS = 16
_NW = _NC * _NS
_B_PER_W = _SEQ // _NW

_CH = 64
_NCH = _B_PER_W // _CH

_mesh = plsc.VectorSubcoreMesh(core_axis_name="c", subcore_axis_name="s")


@functools.partial(
    pl.kernel,
    mesh=_mesh,
    out_type=jax.ShapeDtypeStruct((_SEQ, _D), jnp.float32),
    scratch_types=[
        pltpu.VMEM((_NCH, _CH), jnp.int32),
        pltpu.VMEM((_CH, _D), jnp.float32),
        pltpu.VMEM((_CH, _D), jnp.float32),
        pltpu.SemaphoreType.DMA,
        pltpu.SemaphoreType.DMA,
        pltpu.SemaphoreType.DMA,
        pltpu.SemaphoreType.DMA,
    ],
)
def _sc_gather(table_hbm, idx_hbm, out_hbm, idx_v, buf0, buf1,
               gsem0, gsem1, wsem0, wsem1):
    wid = lax.axis_index("s") * _NC + lax.axis_index("c")
    base = wid * _B_PER_W
    pltpu.sync_copy(idx_hbm.at[wid], idx_v)
    bufs = (buf0, buf1)
    gsems = (gsem0, gsem1)
    wsems = (wsem0, wsem1)

    def start_gather(c):
        return pltpu.async_copy(table_hbm.at[idx_v.at[c]], bufs[c % 2],
                                gsems[c % 2])

    g = [start_gather(0), start_gather(1)]
    for c in range(_NCH):
        b = c % 2
        g[b].wait()
        w = pltpu.async_copy(bufs[b], out_hbm.at[pl.ds(base + c * _CH, _CH)],
                             wsems[b])
        if c + 2 < _NCH:
            w.wait()
            g[b] = start_gather(c + 2)
        else:
            w.wait()


_BLK = 512
_LOG1E4_2_OVER_D = 2.0 * math.log(10000.0) / _D
_NJ2 = 2 * ((_D - 1) // 2)


def _pos_body(out_ref):
    i = pl.program_id(0)
    row = (jnp.float32(i * _BLK)
           + lax.broadcasted_iota(jnp.int32, (_BLK, 1), 0).astype(jnp.float32))
    col = lax.broadcasted_iota(jnp.int32, (1, _D), 1)
    j = jnp.floor_divide(col, 2).astype(jnp.float32)
    live = col < _NJ2
    w = jnp.where(live, jnp.exp(j * (-_LOG1E4_2_OVER_D)), 0.0)
    phase = jnp.where((col % 2 == 1) & live, jnp.float32(math.pi / 2), 0.0)
    out_ref[...] = jnp.sin(row * w + phase)


@functools.lru_cache(maxsize=1)
def _pos_matrix():
    out = pl.pallas_call(
        _pos_body,
        grid=(_SEQ // _BLK,),
        out_specs=pl.BlockSpec((_BLK, _D), lambda i: (i, 0)),
        out_shape=jax.ShapeDtypeStruct((_SEQ, _D), jnp.float32),
    )()
    return jax.block_until_ready(out)


def _add_body(emb_ref, pos_ref, out_ref):
    out_ref[...] = emb_ref[...] + pos_ref[...]


def _add_pos(emb, pos):
    return pl.pallas_call(
        _add_body,
        grid=(_SEQ // _BLK,),
        in_specs=[pl.BlockSpec((_BLK, _D), lambda i: (i, 0)),
                  pl.BlockSpec((_BLK, _D), lambda i: (i, 0))],
        out_specs=pl.BlockSpec((_BLK, _D), lambda i: (i, 0)),
        out_shape=jax.ShapeDtypeStruct((_SEQ, _D), jnp.float32),
    )(emb, pos)


def kernel(x, table):
    idx = x.astype(jnp.int32).reshape(_NW, _NCH, _CH)
    emb = _sc_gather(table, idx)
    return _add_pos(emb, _pos_matrix())


# speedup vs baseline: 2.0273x; 2.0273x over previous
import functools
import math

import jax
import jax.numpy as jnp
from jax import lax
from jax.experimental import pallas as pl
from jax.experimental.pallas import tpu as pltpu
from jax.experimental.pallas import tpu_sc as plsc

_VOCAB = 100000
_D = 512
_SEQ = 8192

_NC = 2
_NS = 16
_NW = _NC * _NS
_B_PER_W = _SEQ // _NW

_CH = 64
_NCH = _B_PER_W // _CH

_mesh = plsc.VectorSubcoreMesh(core_axis_name="c", subcore_axis_name="s")


@functools.partial(
    pl.kernel,
    mesh=_mesh,
    out_type=jax.ShapeDtypeStruct((_SEQ, _D), jnp.float32),
    scratch_types=[
        pltpu.VMEM((_B_PER_W,), jnp.int32),
        pltpu.VMEM((_CH, _D), jnp.float32),
        pltpu.VMEM((_CH, _D), jnp.float32),
        pltpu.VMEM((_CH, _D), jnp.float32),
        pltpu.SemaphoreType.DMA,
        pltpu.SemaphoreType.DMA,
        pltpu.SemaphoreType.DMA,
        pltpu.SemaphoreType.DMA,
        pltpu.SemaphoreType.DMA,
        pltpu.SemaphoreType.DMA,
    ],
)
def _sc_gather(table_hbm, idx_hbm, out_hbm, idx_v, buf0, buf1, buf2,
               gsem0, gsem1, gsem2, wsem0, wsem1, wsem2):
    wid = lax.axis_index("s") * _NC + lax.axis_index("c")
    base = wid * _B_PER_W
    pltpu.sync_copy(idx_hbm.at[pl.ds(base, _B_PER_W)], idx_v)
    bufs = (buf0, buf1, buf2)
    gsems = (gsem0, gsem1, gsem2)
    wsems = (wsem0, wsem1, wsem2)
    nbuf = len(bufs)

    def start_gather(c):
        return pltpu.async_copy(table_hbm.at[idx_v.at[pl.ds(c * _CH, _CH)]],
                                bufs[c % nbuf], gsems[c % nbuf])

    g = [start_gather(c) for c in range(min(nbuf, _NCH))]
    wb = [None] * nbuf
    for c in range(_NCH):
        b = c % nbuf
        g[b].wait()
        wb[b] = pltpu.async_copy(bufs[b],
                                 out_hbm.at[pl.ds(base + c * _CH, _CH)],
                                 wsems[b])
        if c + nbuf < _NCH:
            wb[b].wait()
            g[b] = start_gather(c + nbuf)
    for c in range(max(0, _NCH - nbuf), _NCH):
        wb[c % nbuf].wait()


_BLK = 2048
_NA = 32
_NB = _BLK // _NA
_LOG1E4_2_OVER_D = 2.0 * math.log(10000.0) / _D
_NJ2 = 2 * ((_D - 1) // 2)


def _add_pos_body(emb_ref, out_ref, sb_ref, cb_ref):
    i = pl.program_id(0)
    col = lax.broadcasted_iota(jnp.int32, (1, _D), 1)
    j = jnp.floor_divide(col, 2).astype(jnp.float32)
    live = col < _NJ2
    w = jnp.where(live, jnp.exp(j * (-_LOG1E4_2_OVER_D)), 0.0)
    phase = jnp.where((col % 2 == 1) & live, jnp.float32(math.pi / 2), 0.0)

    @pl.when(i == 0)
    def _init():
        b = lax.broadcasted_iota(jnp.int32, (_NB, 1), 0).astype(jnp.float32)
        zb = b * w
        sb_ref[...] = jnp.sin(zb)
        cb_ref[...] = jnp.cos(zb)

    a = lax.broadcasted_iota(jnp.int32, (_NA, 1), 0).astype(jnp.float32)
    xy = (jnp.float32(i * _BLK) + a * _NB) * w + phase
    sa = jnp.sin(xy)[:, None, :]
    ca = jnp.cos(xy)[:, None, :]
    sb = sb_ref[...][None, :, :]
    cb = cb_ref[...][None, :, :]
    pos = (sa * cb + ca * sb).reshape(_BLK, _D)
    out_ref[...] = emb_ref[...] + pos


def _add_pos(emb):
    return pl.pallas_call(
        _add_pos_body,
        grid=(_SEQ // _BLK,),
        in_specs=[pl.BlockSpec((_BLK, _D), lambda i: (i, 0))],
        out_specs=pl.BlockSpec((_BLK, _D), lambda i: (i, 0)),
        out_shape=jax.ShapeDtypeStruct((_SEQ, _D), jnp.float32),
        scratch_shapes=[pltpu.VMEM((_NB, _D), jnp.float32),
                        pltpu.VMEM((_NB, _D), jnp.float32)],
        input_output_aliases={0: 0},
    )(emb)


def kernel(x, table):
    emb = _sc_gather(table, x.astype(jnp.int32))
    return _add_pos(emb)


# speedup vs baseline: 2.0453x; 1.0088x over previous
import functools
import math

import jax
import jax.numpy as jnp
from jax import lax
from jax.experimental import pallas as pl
from jax.experimental.pallas import tpu as pltpu
from jax.experimental.pallas import tpu_sc as plsc

_VOCAB = 100000
_D = 512
_SEQ = 8192

_NC = 2
_NS = 16
_NW = _NC * _NS
_B_PER_W = _SEQ // _NW

_CH = 32
_NCH = _B_PER_W // _CH

_mesh = plsc.VectorSubcoreMesh(core_axis_name="c", subcore_axis_name="s")


@functools.partial(
    pl.kernel,
    mesh=_mesh,
    out_type=jax.ShapeDtypeStruct((_SEQ, _D), jnp.float32),
    scratch_types=[
        pltpu.VMEM((_B_PER_W,), jnp.int32),
        pltpu.VMEM((_CH, _D), jnp.float32),
        pltpu.VMEM((_CH, _D), jnp.float32),
        pltpu.VMEM((_CH, _D), jnp.float32),
        pltpu.VMEM((_CH, _D), jnp.float32),
        pltpu.VMEM((_CH, _D), jnp.float32),
        pltpu.VMEM((_CH, _D), jnp.float32),
        pltpu.VMEM((_CH, _D), jnp.float32),
        pltpu.SemaphoreType.DMA,
        pltpu.SemaphoreType.DMA,
        pltpu.SemaphoreType.DMA,
        pltpu.SemaphoreType.DMA,
        pltpu.SemaphoreType.DMA,
        pltpu.SemaphoreType.DMA,
        pltpu.SemaphoreType.DMA,
        pltpu.SemaphoreType.DMA,
        pltpu.SemaphoreType.DMA,
        pltpu.SemaphoreType.DMA,
        pltpu.SemaphoreType.DMA,
        pltpu.SemaphoreType.DMA,
        pltpu.SemaphoreType.DMA,
        pltpu.SemaphoreType.DMA,
    ],
)
def _sc_gather(table_hbm, idx_hbm, out_hbm, idx_v, buf0, buf1, buf2, buf3,
               buf4, buf5, buf6, gsem0, gsem1, gsem2, gsem3, gsem4, gsem5,
               gsem6, wsem0, wsem1, wsem2, wsem3, wsem4, wsem5, wsem6):
    wid = lax.axis_index("s") * _NC + lax.axis_index("c")
    base = wid * _B_PER_W
    pltpu.sync_copy(idx_hbm.at[pl.ds(base, _B_PER_W)], idx_v)
    bufs = (buf0, buf1, buf2, buf3, buf4, buf5, buf6)
    gsems = (gsem0, gsem1, gsem2, gsem3, gsem4, gsem5, gsem6)
    wsems = (wsem0, wsem1, wsem2, wsem3, wsem4, wsem5, wsem6)
    nbuf = len(bufs)

    def start_gather(c):
        return pltpu.async_copy(table_hbm.at[idx_v.at[pl.ds(c * _CH, _CH)]],
                                bufs[c % nbuf], gsems[c % nbuf])

    g = [start_gather(c) for c in range(min(nbuf, _NCH))]
    wb = [None] * nbuf
    for c in range(_NCH):
        b = c % nbuf
        g[b].wait()
        wb[b] = pltpu.async_copy(bufs[b],
                                 out_hbm.at[pl.ds(base + c * _CH, _CH)],
                                 wsems[b])
        if c + nbuf < _NCH:
            wb[b].wait()
            g[b] = start_gather(c + nbuf)
    for c in range(max(0, _NCH - nbuf), _NCH):
        wb[c % nbuf].wait()


_BLK = 2048
_NA = 32
_NB = _BLK // _NA
_LOG1E4_2_OVER_D = 2.0 * math.log(10000.0) / _D
_NJ2 = 2 * ((_D - 1) // 2)


def _add_pos_body(emb_ref, out_ref, sb_ref, cb_ref):
    i = pl.program_id(0)
    col = lax.broadcasted_iota(jnp.int32, (1, _D), 1)
    j = jnp.floor_divide(col, 2).astype(jnp.float32)
    live = col < _NJ2
    w = jnp.where(live, jnp.exp(j * (-_LOG1E4_2_OVER_D)), 0.0)
    phase = jnp.where((col % 2 == 1) & live, jnp.float32(math.pi / 2), 0.0)

    @pl.when(i == 0)
    def _init():
        b = lax.broadcasted_iota(jnp.int32, (_NB, 1), 0).astype(jnp.float32)
        zb = b * w
        sb_ref[...] = jnp.sin(zb)
        cb_ref[...] = jnp.cos(zb)

    a = lax.broadcasted_iota(jnp.int32, (_NA, 1), 0).astype(jnp.float32)
    xy = (jnp.float32(i * _BLK) + a * _NB) * w + phase
    sa = jnp.sin(xy)[:, None, :]
    ca = jnp.cos(xy)[:, None, :]
    sb = sb_ref[...][None, :, :]
    cb = cb_ref[...][None, :, :]
    pos = (sa * cb + ca * sb).reshape(_BLK, _D)
    out_ref[...] = emb_ref[...] + pos


def _add_pos(emb):
    return pl.pallas_call(
        _add_pos_body,
        grid=(_SEQ // _BLK,),
        in_specs=[pl.BlockSpec((_BLK, _D), lambda i: (i, 0))],
        out_specs=pl.BlockSpec((_BLK, _D), lambda i: (i, 0)),
        out_shape=jax.ShapeDtypeStruct((_SEQ, _D), jnp.float32),
        scratch_shapes=[pltpu.VMEM((_NB, _D), jnp.float32),
                        pltpu.VMEM((_NB, _D), jnp.float32)],
        input_output_aliases={0: 0},
    )(emb)


def kernel(x, table):
    emb = _sc_gather(table, x.astype(jnp.int32))
    return _add_pos(emb)


# speedup vs baseline: 2.1299x; 1.0414x over previous
import functools
import math

import jax
import jax.numpy as jnp
from jax import lax
from jax.experimental import pallas as pl
from jax.experimental.pallas import tpu as pltpu
from jax.experimental.pallas import tpu_sc as plsc

_VOCAB = 100000
_D = 512
_SEQ = 8192

_NC = 2
_NS = 16
_NW = _NC * _NS
_B_PER_W = _SEQ // _NW

_CH = 32
_NCH = _B_PER_W // _CH

_mesh = plsc.VectorSubcoreMesh(core_axis_name="c", subcore_axis_name="s")


@functools.partial(
    pl.kernel,
    mesh=_mesh,
    out_type=jax.ShapeDtypeStruct((_SEQ, _D), jnp.float32),
    scratch_types=[
        pltpu.VMEM((_B_PER_W,), jnp.int32),
        pltpu.VMEM((_CH, _D), jnp.float32),
        pltpu.VMEM((_CH, _D), jnp.float32),
        pltpu.VMEM((_CH, _D), jnp.float32),
        pltpu.VMEM((_CH, _D), jnp.float32),
        pltpu.VMEM((_CH, _D), jnp.float32),
        pltpu.VMEM((_CH, _D), jnp.float32),
        pltpu.VMEM((_CH, _D), jnp.float32),
        pltpu.SemaphoreType.DMA,
        pltpu.SemaphoreType.DMA,
        pltpu.SemaphoreType.DMA,
        pltpu.SemaphoreType.DMA,
        pltpu.SemaphoreType.DMA,
        pltpu.SemaphoreType.DMA,
        pltpu.SemaphoreType.DMA,
        pltpu.SemaphoreType.DMA,
        pltpu.SemaphoreType.DMA,
        pltpu.SemaphoreType.DMA,
        pltpu.SemaphoreType.DMA,
        pltpu.SemaphoreType.DMA,
        pltpu.SemaphoreType.DMA,
        pltpu.SemaphoreType.DMA,
    ],
)
def _sc_gather(table_hbm, idx_hbm, out_hbm, idx_v, buf0, buf1, buf2, buf3,
               buf4, buf5, buf6, gsem0, gsem1, gsem2, gsem3, gsem4, gsem5,
               gsem6, wsem0, wsem1, wsem2, wsem3, wsem4, wsem5, wsem6):
    wid = lax.axis_index("s") * _NC + lax.axis_index("c")
    base = wid * _B_PER_W
    pltpu.sync_copy(idx_hbm.at[pl.ds(base, _B_PER_W)], idx_v)
    bufs = (buf0, buf1, buf2, buf3, buf4, buf5, buf6)
    gsems = (gsem0, gsem1, gsem2, gsem3, gsem4, gsem5, gsem6)
    wsems = (wsem0, wsem1, wsem2, wsem3, wsem4, wsem5, wsem6)
    nbuf = len(bufs)

    def start_gather(c):
        return pltpu.async_copy(table_hbm.at[idx_v.at[pl.ds(c * _CH, _CH)]],
                                bufs[c % nbuf], gsems[c % nbuf])

    g = [start_gather(c) for c in range(min(nbuf, _NCH))]
    wb = [None] * nbuf
    for c in range(_NCH):
        b = c % nbuf
        g[b].wait()
        wb[b] = pltpu.async_copy(bufs[b],
                                 out_hbm.at[pl.ds(base + c * _CH, _CH)],
                                 wsems[b])
        if c + nbuf < _NCH:
            wb[b].wait()
            g[b] = start_gather(c + nbuf)
    for c in range(max(0, _NCH - nbuf), _NCH):
        wb[c % nbuf].wait()


_BLK = 4096
_NA = 64
_NB = _BLK // _NA
_LOG1E4_2_OVER_D = 2.0 * math.log(10000.0) / _D
_NJ2 = 2 * ((_D - 1) // 2)


def _add_pos_body(emb_ref, out_ref, sb_ref, cb_ref):
    i = pl.program_id(0)
    col = lax.broadcasted_iota(jnp.int32, (1, _D), 1)
    j = jnp.floor_divide(col, 2).astype(jnp.float32)
    live = col < _NJ2
    w = jnp.where(live, jnp.exp(j * (-_LOG1E4_2_OVER_D)), 0.0)
    phase = jnp.where((col % 2 == 1) & live, jnp.float32(math.pi / 2), 0.0)

    @pl.when(i == 0)
    def _init():
        b = lax.broadcasted_iota(jnp.int32, (_NB, 1), 0).astype(jnp.float32)
        zb = b * w
        sb_ref[...] = jnp.sin(zb)
        cb_ref[...] = jnp.cos(zb)

    a = lax.broadcasted_iota(jnp.int32, (_NA, 1), 0).astype(jnp.float32)
    xy = (jnp.float32(i * _BLK) + a * _NB) * w + phase
    sa = jnp.sin(xy)[:, None, :]
    ca = jnp.cos(xy)[:, None, :]
    sb = sb_ref[...][None, :, :]
    cb = cb_ref[...][None, :, :]
    pos = (sa * cb + ca * sb).reshape(_BLK, _D)
    out_ref[...] = emb_ref[...] + pos


def _add_pos(emb):
    return pl.pallas_call(
        _add_pos_body,
        grid=(_SEQ // _BLK,),
        in_specs=[pl.BlockSpec((_BLK, _D), lambda i: (i, 0))],
        out_specs=pl.BlockSpec((_BLK, _D), lambda i: (i, 0)),
        out_shape=jax.ShapeDtypeStruct((_SEQ, _D), jnp.float32),
        scratch_shapes=[pltpu.VMEM((_NB, _D), jnp.float32),
                        pltpu.VMEM((_NB, _D), jnp.float32)],
        input_output_aliases={0: 0},
    )(emb)


def kernel(x, table):
    emb = _sc_gather(table, x.astype(jnp.int32))
    return _add_pos(emb)
